# baseline (speedup 1.0000x reference)
import functools

import jax
import jax.numpy as jnp
from jax import lax
from jax.experimental import pallas as pl
from jax.experimental.pallas import tpu as pltpu
from jax.experimental.pallas import tpu_sc as plsc

VOCAB = 100000
EMBED = 32
BATCH = 1024
CTX = 20

NC, NS = 2, 16
NW = NC * NS
BPW = BATCH // NW
IDX_PER_W = BPW * CTX
IDX_CHUNK = 128
N_CHUNKS = IDX_PER_W // IDX_CHUNK

@functools.cache
def _make_sc_pool():
    mesh = plsc.VectorSubcoreMesh(core_axis_name="c", subcore_axis_name="s")
    return pl.kernel(
        _sc_pool_body,
        out_type=jax.ShapeDtypeStruct((BATCH, EMBED), jnp.float32),
        mesh=mesh,
        scratch_types=[
            pltpu.VMEM((N_CHUNKS, IDX_CHUNK), jnp.int32),
            pltpu.VMEM((IDX_PER_W, EMBED), jnp.float32),
            pltpu.VMEM((BPW, EMBED), jnp.float32),
            pltpu.SemaphoreType.DMA,
        ],
        compiler_params=pltpu.CompilerParams(use_tc_tiling_on_sc=False),
    )


def _sc_pool_body(table_hbm, idx_hbm, out_hbm, idx_v, rows_v, pooled_v, sem):
    wid = lax.axis_index("s") * NC + lax.axis_index("c")
    pltpu.sync_copy(idx_hbm.at[wid], idx_v)
    copies = []
    for j in range(N_CHUNKS):
        copies.append(
            pltpu.async_copy(
                table_hbm.at[idx_v.at[j]],
                rows_v.at[pl.ds(j * IDX_CHUNK, IDX_CHUNK)],
                sem,
            )
        )
    for c in copies:
        c.wait()

    inv = jnp.float32(1.0 / CTX)

    def body_e(e, _):
        def body_r(r, acc):
            a0, a1 = acc
            row = e * CTX + r
            return (a0 + rows_v[row, pl.ds(0, 16)],
                    a1 + rows_v[row, pl.ds(16, 16)])

        a0, a1 = lax.fori_loop(
            0, CTX, body_r,
            (jnp.zeros(16, jnp.float32), jnp.zeros(16, jnp.float32)),
        )
        pooled_v[e, pl.ds(0, 16)] = a0 * inv
        pooled_v[e, pl.ds(16, 16)] = a1 * inv
        return 0

    lax.fori_loop(0, BPW, body_e, 0)
    pltpu.sync_copy(pooled_v, out_hbm.at[pl.ds(wid * BPW, BPW)])


N_TILE = 2048


def _tc_matmul_body(pooled_ref, w_ref, b_ref, out_ref):
    out_ref[...] = lax.dot_general(
        pooled_ref[...], w_ref[...],
        (((1,), (1,)), ((), ())),
        preferred_element_type=jnp.float32,
    ) + b_ref[...]


def _tc_matmul(pooled, W, b2d):
    n_blocks = (VOCAB + N_TILE - 1) // N_TILE
    return pl.pallas_call(
        _tc_matmul_body,
        grid=(n_blocks,),
        in_specs=[
            pl.BlockSpec((BATCH, EMBED), lambda j: (0, 0)),
            pl.BlockSpec((N_TILE, EMBED), lambda j: (j, 0)),
            pl.BlockSpec((1, N_TILE), lambda j: (0, j)),
        ],
        out_specs=pl.BlockSpec((BATCH, N_TILE), lambda j: (0, j)),
        out_shape=jax.ShapeDtypeStruct((BATCH, VOCAB), jnp.float32),
        compiler_params=pltpu.CompilerParams(
            dimension_semantics=("arbitrary",),
        ),
    )(pooled, W, b2d)


def kernel(inputs, emb_table, W, b):
    idx = inputs.astype(jnp.int32).reshape(NW, N_CHUNKS, IDX_CHUNK)
    pooled = _make_sc_pool()(emb_table, idx)
    return _tc_matmul(pooled, W, b.reshape(1, VOCAB))


# speedup vs baseline: 3.1049x; 3.1049x over previous
import functools

import jax
import jax.numpy as jnp
from jax import lax
from jax.experimental import pallas as pl
from jax.experimental.pallas import tpu as pltpu
from jax.experimental.pallas import tpu_sc as plsc

VOCAB = 100000
EMBED = 32
BATCH = 1024
CTX = 20

NC, NS = 2, 16
NW = NC * NS
BPW = BATCH // NW
IDX_PER_W = BPW * CTX
IDX_CHUNK = 128
N_CHUNKS = IDX_PER_W // IDX_CHUNK

LANE_PAD = 128


@functools.cache
def _make_sc_pool():
    mesh = plsc.VectorSubcoreMesh(core_axis_name="c", subcore_axis_name="s")
    return pl.kernel(
        _sc_pool_body,
        out_type=jax.ShapeDtypeStruct((BATCH, EMBED), jnp.float32),
        mesh=mesh,
        scratch_types=[
            pltpu.VMEM((N_CHUNKS, IDX_CHUNK), jnp.int32),
            pltpu.VMEM((IDX_PER_W, LANE_PAD), jnp.float32),
            pltpu.VMEM((BPW, EMBED), jnp.float32),
            pltpu.SemaphoreType.DMA,
        ],
        compiler_params=pltpu.CompilerParams(use_tc_tiling_on_sc=True),
    )


def _sc_pool_body(table_hbm, idx_hbm, out_hbm, idx_v, rows_v, pooled_v, sem):
    wid = lax.axis_index("s") * NC + lax.axis_index("c")
    pltpu.sync_copy(idx_hbm.at[wid], idx_v)
    copies = []
    for j in range(N_CHUNKS):
        copies.append(
            pltpu.async_copy(
                table_hbm.at[idx_v.at[j]],
                rows_v.at[pl.ds(j * IDX_CHUNK, IDX_CHUNK)],
                sem,
            )
        )
    for c in copies:
        c.wait()

    inv = jnp.float32(1.0 / CTX)

    def body_e(e, _):
        def body_r(r, acc):
            a0, a1 = acc
            row = e * CTX + r
            return (a0 + rows_v[row, pl.ds(0, 16)],
                    a1 + rows_v[row, pl.ds(16, 16)])

        a0, a1 = lax.fori_loop(
            0, CTX, body_r,
            (jnp.zeros(16, jnp.float32), jnp.zeros(16, jnp.float32)),
        )
        pooled_v[e, pl.ds(0, 16)] = a0 * inv
        pooled_v[e, pl.ds(16, 16)] = a1 * inv
        return 0

    lax.fori_loop(0, BPW, body_e, 0)
    pltpu.sync_copy(pooled_v, out_hbm.at[pl.ds(wid * BPW, BPW)])


T_TILE = 4096


def _tc_transpose_body(emb_t_ref, eye_ref, out_ref):
    out_ref[...] = lax.dot_general(
        emb_t_ref[...], eye_ref[...],
        (((0,), (0,)), ((), ())),
        preferred_element_type=jnp.float32,
    )


def _tc_transpose_pad(emb_t, eye):
    nb = (VOCAB + T_TILE - 1) // T_TILE
    return pl.pallas_call(
        _tc_transpose_body,
        grid=(nb,),
        in_specs=[
            pl.BlockSpec((EMBED, T_TILE), lambda j: (0, j)),
            pl.BlockSpec((EMBED, LANE_PAD), lambda j: (0, 0)),
        ],
        out_specs=pl.BlockSpec((T_TILE, LANE_PAD), lambda j: (j, 0)),
        out_shape=jax.ShapeDtypeStruct((VOCAB, LANE_PAD), jnp.float32),
        compiler_params=pltpu.CompilerParams(
            dimension_semantics=("arbitrary",),
        ),
    )(emb_t, eye)


N_TILE = 2048


def _tc_matmul_body(pt_ref, wt_ref, b_ref, out_ref):
    acc = lax.dot_general(
        wt_ref[...], pt_ref[...],
        (((0,), (0,)), ((), ())),
        preferred_element_type=jnp.float32,
    )
    out_ref[...] = acc + jnp.transpose(b_ref[...])


def _tc_matmul_t(pooled_t, wt, b2d):
    n_blocks = (VOCAB + N_TILE - 1) // N_TILE
    return pl.pallas_call(
        _tc_matmul_body,
        grid=(n_blocks,),
        in_specs=[
            pl.BlockSpec((EMBED, BATCH), lambda j: (0, 0)),
            pl.BlockSpec((EMBED, N_TILE), lambda j: (0, j)),
            pl.BlockSpec((1, N_TILE), lambda j: (0, j)),
        ],
        out_specs=pl.BlockSpec((N_TILE, BATCH), lambda j: (j, 0)),
        out_shape=jax.ShapeDtypeStruct((VOCAB, BATCH), jnp.float32),
        compiler_params=pltpu.CompilerParams(
            dimension_semantics=("arbitrary",),
        ),
    )(pooled_t, wt, b2d)


def kernel(inputs, emb_table, W, b):
    idx = inputs.astype(jnp.int32).reshape(NW, N_CHUNKS, IDX_CHUNK)
    table128 = _tc_transpose_pad(
        emb_table.T, jnp.eye(EMBED, LANE_PAD, dtype=jnp.float32))
    pooled = _make_sc_pool()(table128, idx)
    out_t = _tc_matmul_t(pooled.T, W.T, b.reshape(1, VOCAB))
    return out_t.T
